# baseline (speedup 1.0000x reference)
import functools

import jax
import jax.numpy as jnp
from jax import lax
from jax.experimental import pallas as pl
from jax.experimental.pallas import tpu as pltpu
from jax.experimental.pallas import tpu_sc as plsc

_NCLASS = 100000
_BATCH = 1024
_LANES = 16
_TILE = 128
_WIN = 2 * _TILE
_LASTA = (_NCLASS // _TILE - 1) * _TILE
_TAIL = _NCLASS - _TILE
_NW = 32
_BPW = _BATCH // _NW

_BB = 64


_SPLITS = [0, 25088, 50176, 75264, _NCLASS]


def _lse_body(tgt_sref, x_ref, lse_ref, tile_ref):
    x = x_ref[...]
    parts_m = [jnp.max(x[:, a:b], axis=1) for a, b in zip(_SPLITS, _SPLITS[1:])]
    m = jnp.maximum(jnp.maximum(parts_m[0], parts_m[1]),
                    jnp.maximum(parts_m[2], parts_m[3]))
    mc = m[:, None]
    parts_s = [jnp.sum(jnp.exp(x[:, a:b] - mc), axis=1)
               for a, b in zip(_SPLITS, _SPLITS[1:])]
    s = (parts_s[0] + parts_s[1]) + (parts_s[2] + parts_s[3])
    lse_ref[...] = (m + jnp.log(s))[None, None, :]
    tile_ref[:, _TILE:] = x_ref[:, _TAIL:]
    pid = pl.program_id(0)
    for b in range(_BB):
        t = tgt_sref[pid * _BB + b]
        start = jnp.minimum((t // _TILE) * _TILE, _LASTA)
        start = pl.multiple_of(start, _TILE)
        tile_ref[b, :_TILE] = x_ref[b, pl.ds(start, _TILE)]


def _lse_and_tiles(x, target):
    grid_spec = pltpu.PrefetchScalarGridSpec(
        num_scalar_prefetch=1,
        grid=(_BATCH // _BB,),
        in_specs=[pl.BlockSpec((_BB, _NCLASS), lambda i, tgt: (i, 0))],
        out_specs=[
            pl.BlockSpec((1, 1, _BB), lambda i, tgt: (i, 0, 0)),
            pl.BlockSpec((_BB, _WIN), lambda i, tgt: (i, 0)),
        ],
    )
    lse3, tiles = pl.pallas_call(
        _lse_body,
        grid_spec=grid_spec,
        out_shape=[
            jax.ShapeDtypeStruct((_BATCH // _BB, 1, _BB), jnp.float32),
            jax.ShapeDtypeStruct((_BATCH, _WIN), jnp.float32),
        ],
    )(target, x)
    return lse3.reshape(_BATCH), tiles.reshape(_BATCH * _WIN)


@functools.cache
def _make_sc_gather_sub():
    mesh = plsc.VectorSubcoreMesh(core_axis_name="c", subcore_axis_name="s")

    @functools.partial(
        pl.kernel,
        mesh=mesh,
        out_type=jax.ShapeDtypeStruct((_BATCH,), jnp.float32),
        scratch_types=[
            pltpu.VMEM((_BPW,), jnp.int32),
            pltpu.VMEM((_BPW,), jnp.int32),
            pltpu.VMEM((_BPW,), jnp.float32),
            pltpu.VMEM((_BPW,), jnp.float32),
            pltpu.VMEM((_BPW,), jnp.float32),
            pltpu.SemaphoreType.DMA,
        ],
    )
    def _sc_gather_sub(table_hbm, tgt_hbm, lse_hbm, out_hbm,
                       tgt_v, idx_v, val_v, lse_v, out_v, sem):
        wid = lax.axis_index("s") * 2 + lax.axis_index("c")
        base = wid * _BPW
        pltpu.sync_copy(tgt_hbm.at[pl.ds(base, _BPW)], tgt_v)
        pltpu.sync_copy(lse_hbm.at[pl.ds(base, _BPW)], lse_v)
        for g in range(_BPW // _LANES):
            sl = pl.ds(g * _LANES, _LANES)
            t = tgt_v[sl]
            start = jnp.minimum((t >> 7) << 7, _LASTA)
            lane = jnp.where(t >= _TAIL, _TILE + (t - _TAIL), t - start)
            row = base + g * _LANES + lax.iota(jnp.int32, _LANES)
            idx_v[sl] = row * _WIN + lane
        pltpu.async_copy(table_hbm.at[idx_v], val_v, sem).wait()
        for g in range(_BPW // _LANES):
            sl = pl.ds(g * _LANES, _LANES)
            out_v[sl] = lse_v[sl] - val_v[sl]
        pltpu.sync_copy(out_v, out_hbm.at[pl.ds(base, _BPW)])

    return _sc_gather_sub


def kernel(input, target):
    lse, table = _lse_and_tiles(input, target)
    return _make_sc_gather_sub()(table, target, lse)


# speedup vs baseline: 1.0030x; 1.0030x over previous
import functools

import jax
import jax.numpy as jnp
from jax import lax
from jax.experimental import pallas as pl
from jax.experimental.pallas import tpu as pltpu
from jax.experimental.pallas import tpu_sc as plsc

_NCLASS = 100000
_BATCH = 1024
_LANES = 16
_TILE = 128
_WIN = 2 * _TILE
_LASTA = (_NCLASS // _TILE - 1) * _TILE
_TAIL = _NCLASS - _TILE
_NW = 32
_BPW = _BATCH // _NW

_BB = 32


_SPLITS = [0, 25088, 50176, 75264, _NCLASS]


def _lse_body(tgt_sref, x_ref, lse_ref, tile_ref):
    x = x_ref[...]
    parts_m = [jnp.max(x[:, a:b], axis=1) for a, b in zip(_SPLITS, _SPLITS[1:])]
    m = jnp.maximum(jnp.maximum(parts_m[0], parts_m[1]),
                    jnp.maximum(parts_m[2], parts_m[3]))
    mc = m[:, None]
    parts_s = [jnp.sum(jnp.exp(x[:, a:b] - mc), axis=1)
               for a, b in zip(_SPLITS, _SPLITS[1:])]
    s = (parts_s[0] + parts_s[1]) + (parts_s[2] + parts_s[3])
    lse_ref[...] = (m + jnp.log(s))[None, None, :]
    tile_ref[:, _TILE:] = x_ref[:, _TAIL:]
    pid = pl.program_id(0)
    for b in range(_BB):
        t = tgt_sref[pid * _BB + b]
        start = jnp.minimum((t // _TILE) * _TILE, _LASTA)
        start = pl.multiple_of(start, _TILE)
        tile_ref[b, :_TILE] = x_ref[b, pl.ds(start, _TILE)]


def _lse_and_tiles(x, target):
    grid_spec = pltpu.PrefetchScalarGridSpec(
        num_scalar_prefetch=1,
        grid=(_BATCH // _BB,),
        in_specs=[pl.BlockSpec((_BB, _NCLASS), lambda i, tgt: (i, 0))],
        out_specs=[
            pl.BlockSpec((1, 1, _BB), lambda i, tgt: (i, 0, 0)),
            pl.BlockSpec((_BB, _WIN), lambda i, tgt: (i, 0)),
        ],
    )
    lse3, tiles = pl.pallas_call(
        _lse_body,
        grid_spec=grid_spec,
        out_shape=[
            jax.ShapeDtypeStruct((_BATCH // _BB, 1, _BB), jnp.float32),
            jax.ShapeDtypeStruct((_BATCH, _WIN), jnp.float32),
        ],
    )(target, x)
    return lse3.reshape(_BATCH), tiles.reshape(_BATCH * _WIN)


@functools.cache
def _make_sc_gather_sub():
    mesh = plsc.VectorSubcoreMesh(core_axis_name="c", subcore_axis_name="s")

    @functools.partial(
        pl.kernel,
        mesh=mesh,
        out_type=jax.ShapeDtypeStruct((_BATCH,), jnp.float32),
        scratch_types=[
            pltpu.VMEM((_BPW,), jnp.int32),
            pltpu.VMEM((_BPW,), jnp.int32),
            pltpu.VMEM((_BPW,), jnp.float32),
            pltpu.VMEM((_BPW,), jnp.float32),
            pltpu.VMEM((_BPW,), jnp.float32),
            pltpu.SemaphoreType.DMA,
        ],
    )
    def _sc_gather_sub(table_hbm, tgt_hbm, lse_hbm, out_hbm,
                       tgt_v, idx_v, val_v, lse_v, out_v, sem):
        wid = lax.axis_index("s") * 2 + lax.axis_index("c")
        base = wid * _BPW
        pltpu.sync_copy(tgt_hbm.at[pl.ds(base, _BPW)], tgt_v)
        pltpu.sync_copy(lse_hbm.at[pl.ds(base, _BPW)], lse_v)
        for g in range(_BPW // _LANES):
            sl = pl.ds(g * _LANES, _LANES)
            t = tgt_v[sl]
            start = jnp.minimum((t >> 7) << 7, _LASTA)
            lane = jnp.where(t >= _TAIL, _TILE + (t - _TAIL), t - start)
            row = base + g * _LANES + lax.iota(jnp.int32, _LANES)
            idx_v[sl] = row * _WIN + lane
        pltpu.async_copy(table_hbm.at[idx_v], val_v, sem).wait()
        for g in range(_BPW // _LANES):
            sl = pl.ds(g * _LANES, _LANES)
            out_v[sl] = lse_v[sl] - val_v[sl]
        pltpu.sync_copy(out_v, out_hbm.at[pl.ds(base, _BPW)])

    return _sc_gather_sub


def kernel(input, target):
    lse, table = _lse_and_tiles(input, target)
    return _make_sc_gather_sub()(table, target, lse)


# speedup vs baseline: 1.0058x; 1.0029x over previous
import functools

import jax
import jax.numpy as jnp
from jax import lax
from jax.experimental import pallas as pl
from jax.experimental.pallas import tpu as pltpu
from jax.experimental.pallas import tpu_sc as plsc

_NCLASS = 100000
_BATCH = 1024
_LANES = 16
_TILE = 128
_WIN = 2 * _TILE
_LASTA = (_NCLASS // _TILE - 1) * _TILE
_TAIL = _NCLASS - _TILE
_NW = 32
_BPW = _BATCH // _NW

_BB = 32


_NCHUNKS = 8
_CW = (_NCLASS // (_NCHUNKS * 128)) * 128
_BOUNDS = [i * _CW for i in range(_NCHUNKS)] + [_NCLASS]


def _lse_body(tgt_sref, x_ref, lse_ref, tile_ref):
    parts_m = [jnp.max(x_ref[:, a:b], axis=1)
               for a, b in zip(_BOUNDS, _BOUNDS[1:])]
    m = parts_m[0]
    for pm in parts_m[1:]:
        m = jnp.maximum(m, pm)
    mc = m[:, None]
    parts_s = [jnp.sum(jnp.exp(x_ref[:, a:b] - mc), axis=1)
               for a, b in zip(_BOUNDS, _BOUNDS[1:])]
    s = parts_s[0]
    for ps in parts_s[1:]:
        s = s + ps
    lse_ref[...] = (m + jnp.log(s))[None, None, :]
    tile_ref[:, _TILE:] = x_ref[:, _TAIL:]
    pid = pl.program_id(0)
    for b in range(_BB):
        t = tgt_sref[pid * _BB + b]
        start = jnp.minimum((t // _TILE) * _TILE, _LASTA)
        start = pl.multiple_of(start, _TILE)
        tile_ref[b, :_TILE] = x_ref[b, pl.ds(start, _TILE)]


def _lse_and_tiles(x, target):
    grid_spec = pltpu.PrefetchScalarGridSpec(
        num_scalar_prefetch=1,
        grid=(_BATCH // _BB,),
        in_specs=[pl.BlockSpec((_BB, _NCLASS), lambda i, tgt: (i, 0))],
        out_specs=[
            pl.BlockSpec((1, 1, _BB), lambda i, tgt: (i, 0, 0)),
            pl.BlockSpec((_BB, _WIN), lambda i, tgt: (i, 0)),
        ],
    )
    lse3, tiles = pl.pallas_call(
        _lse_body,
        grid_spec=grid_spec,
        out_shape=[
            jax.ShapeDtypeStruct((_BATCH // _BB, 1, _BB), jnp.float32),
            jax.ShapeDtypeStruct((_BATCH, _WIN), jnp.float32),
        ],
    )(target, x)
    return lse3.reshape(_BATCH), tiles.reshape(_BATCH * _WIN)


@functools.cache
def _make_sc_gather_sub():
    mesh = plsc.VectorSubcoreMesh(core_axis_name="c", subcore_axis_name="s")

    @functools.partial(
        pl.kernel,
        mesh=mesh,
        out_type=jax.ShapeDtypeStruct((_BATCH,), jnp.float32),
        scratch_types=[
            pltpu.VMEM((_BPW,), jnp.int32),
            pltpu.VMEM((_BPW,), jnp.int32),
            pltpu.VMEM((_BPW,), jnp.float32),
            pltpu.VMEM((_BPW,), jnp.float32),
            pltpu.VMEM((_BPW,), jnp.float32),
            pltpu.SemaphoreType.DMA,
        ],
    )
    def _sc_gather_sub(table_hbm, tgt_hbm, lse_hbm, out_hbm,
                       tgt_v, idx_v, val_v, lse_v, out_v, sem):
        wid = lax.axis_index("s") * 2 + lax.axis_index("c")
        base = wid * _BPW
        pltpu.sync_copy(tgt_hbm.at[pl.ds(base, _BPW)], tgt_v)
        pltpu.sync_copy(lse_hbm.at[pl.ds(base, _BPW)], lse_v)
        for g in range(_BPW // _LANES):
            sl = pl.ds(g * _LANES, _LANES)
            t = tgt_v[sl]
            start = jnp.minimum((t >> 7) << 7, _LASTA)
            lane = jnp.where(t >= _TAIL, _TILE + (t - _TAIL), t - start)
            row = base + g * _LANES + lax.iota(jnp.int32, _LANES)
            idx_v[sl] = row * _WIN + lane
        pltpu.async_copy(table_hbm.at[idx_v], val_v, sem).wait()
        for g in range(_BPW // _LANES):
            sl = pl.ds(g * _LANES, _LANES)
            out_v[sl] = lse_v[sl] - val_v[sl]
        pltpu.sync_copy(out_v, out_hbm.at[pl.ds(base, _BPW)])

    return _sc_gather_sub


def kernel(input, target):
    lse, table = _lse_and_tiles(input, target)
    return _make_sc_gather_sub()(table, target, lse)
